# baseline (speedup 1.0000x reference)
import functools

import jax
import jax.numpy as jnp
from jax import lax
from jax.experimental import pallas as pl
from jax.experimental.pallas import tpu as pltpu
from jax.experimental.pallas import tpu_sc as plsc

_INTERPRET = False

B, N, K = 4, 4096, 30
KP = 32
SEM_IN = 128
INS_IN = 128
INS_OUT = 5
SEM_OUT = 13
BN = B * N

RB = 512
CA = 512

_PREC = None


def _embed_body(fs_ref, fi_ref, wa_ref, ba_ref, wi_ref, bi_ref, e_ref):
    fs = fs_ref[0]
    fi = fi_ref[0]
    ad = jnp.maximum(
        jax.lax.dot_general(wa_ref[...], fs, (((1,), (0,)), ((), ())),
                            preferred_element_type=jnp.float32, precision=_PREC)
        + ba_ref[...], 0.0)
    fsins = fi + ad
    e = jax.lax.dot_general(wi_ref[...], fsins, (((1,), (0,)), ((), ())),
                            preferred_element_type=jnp.float32, precision=_PREC)
    e_ref[0] = e + bi_ref[...]


def _embed(f_sem, f_ins, W_adapt, b_adapt, W_ins, b_ins):
    return pl.pallas_call(
        _embed_body,
        grid=(B, N // CA),
        in_specs=[
            pl.BlockSpec((1, SEM_IN, CA), lambda b, j: (b, 0, j)),
            pl.BlockSpec((1, INS_IN, CA), lambda b, j: (b, 0, j)),
            pl.BlockSpec((INS_IN, SEM_IN), lambda b, j: (0, 0)),
            pl.BlockSpec((INS_IN, 1), lambda b, j: (0, 0)),
            pl.BlockSpec((INS_OUT, INS_IN), lambda b, j: (0, 0)),
            pl.BlockSpec((INS_OUT, 1), lambda b, j: (0, 0)),
        ],
        out_specs=pl.BlockSpec((1, INS_OUT, CA), lambda b, j: (b, 0, j)),
        out_shape=jax.ShapeDtypeStruct((B, INS_OUT, N), jnp.float32),
        interpret=_INTERPRET,
    )(f_sem, f_ins, W_adapt, b_adapt, W_ins, b_ins)


def _topk_body(eT_ref, e_ref, idx_ref):
    eq = eT_ref[0]
    ef = e_ref[0]
    sq = jnp.sum(ef * ef, axis=0, keepdims=True)
    sq_q = jnp.sum(eq * eq, axis=1, keepdims=True)
    inner = jax.lax.dot_general(eq, ef, (((1,), (0,)), ((), ())),
                                preferred_element_type=jnp.float32,
                                precision=_PREC)
    score = (sq_q - 2.0 * inner) + sq
    lanes = jax.lax.broadcasted_iota(jnp.int32, (RB, N), 1)
    j = pl.program_id(0)
    idx0 = (jax.lax.broadcasted_iota(jnp.int32, (RB, 1), 0) + j * RB)
    idx_ref[0, :, pl.ds(0, 1)] = idx0
    score = jnp.where(lanes == idx0, jnp.inf, score)
    for k in range(1, K):
        idx = jnp.argmin(score, axis=1, keepdims=True).astype(jnp.int32)
        idx_ref[0, :, pl.ds(k, 1)] = idx
        if k < K - 1:
            score = jnp.where(lanes == idx, jnp.inf, score)
    idx_ref[0, :, pl.ds(K, 1)] = idx0
    idx_ref[0, :, pl.ds(K + 1, 1)] = idx0


def _topk(e_T_b, e_ins_b):
    return pl.pallas_call(
        _topk_body,
        grid=(N // RB,),
        in_specs=[
            pl.BlockSpec((1, RB, INS_OUT), lambda j: (0, j, 0)),
            pl.BlockSpec((1, INS_OUT, N), lambda j: (0, 0, 0)),
        ],
        out_specs=pl.BlockSpec((1, RB, KP), lambda j: (0, j, 0)),
        out_shape=jax.ShapeDtypeStruct((1, N, KP), jnp.int32),
        interpret=_INTERPRET,
    )(e_T_b, e_ins_b)


NW = 32
QW = N // NW
QC = 4
NCHUNK = QW // QC


def _gather_max_body(idx_hbm, table_hbm, out_hbm,
                     idx0, idx1, rows0, rows1, out_v, sem0, sem1):
    wid = lax.axis_index("s") * 2 + lax.axis_index("c")
    qbase = wid * QW
    idx_v = (idx0, idx1)
    rows_v = (rows0, rows1)
    sems = (sem0, sem1)

    def issue(t, slot):
        off = (qbase + t * QC) * KP
        pltpu.sync_copy(idx_hbm.at[pl.ds(off, QC * KP)], idx_v[slot])
        pltpu.async_copy(table_hbm.at[idx_v[slot]], rows_v[slot], sems[slot])

    def compute(t, slot):
        pltpu.make_async_copy(table_hbm.at[idx_v[slot]], rows_v[slot],
                              sems[slot]).wait()
        rv = rows_v[slot]
        for q in range(QC):
            r0 = q * KP

            def body(r, accs):
                rr = r0 + r * 4
                for d in range(4):
                    accs = tuple(
                        jnp.maximum(accs[j], rv[rr + d, pl.ds(j * 16, 16)])
                        for j in range(8))
                return accs

            accs = tuple(rv[r0, pl.ds(j * 16, 16)] for j in range(8))
            accs = tuple(
                jnp.maximum(accs[j], rv[r0 + 1, pl.ds(j * 16, 16)])
                for j in range(8))
            accs = tuple(
                jnp.maximum(accs[j], rv[r0 + 2, pl.ds(j * 16, 16)])
                for j in range(8))
            accs = tuple(
                jnp.maximum(accs[j], rv[r0 + 3, pl.ds(j * 16, 16)])
                for j in range(8))
            accs = lax.fori_loop(1, KP // 4, body, accs)
            qq = t * QC + q
            for j in range(8):
                out_v[qq, pl.ds(j * 16, 16)] = accs[j]

    issue(0, 0)

    def pair(tt, _):
        t0 = tt * 2
        issue(t0 + 1, 1)
        compute(t0, 0)

        @pl.when(tt < NCHUNK // 2 - 1)
        def _():
            issue(t0 + 2, 0)

        compute(t0 + 1, 1)
        return 0

    lax.fori_loop(0, NCHUNK // 2, pair, 0)
    pltpu.sync_copy(out_v, out_hbm.at[pl.ds(qbase, QW)])


@functools.cache
def _gather_max_fn():
    return pl.kernel(
        _gather_max_body,
        out_type=jax.ShapeDtypeStruct((N, SEM_IN), jnp.float32),
        mesh=plsc.VectorSubcoreMesh(core_axis_name="c", subcore_axis_name="s",
                                    num_cores=2, num_subcores=16),
        scratch_types=[
            pltpu.VMEM((QC * KP,), jnp.int32),
            pltpu.VMEM((QC * KP,), jnp.int32),
            pltpu.VMEM((QC * KP, SEM_IN), jnp.float32),
            pltpu.VMEM((QC * KP, SEM_IN), jnp.float32),
            pltpu.VMEM((QW, SEM_IN), jnp.float32),
            pltpu.SemaphoreType.DMA,
            pltpu.SemaphoreType.DMA,
        ],
    )


def _gather_max(idx_flat, table):
    return _gather_max_fn()(idx_flat, table)


def _sempred_body(x_ref, w_ref, b_ref, o_ref):
    o_ref[...] = jax.lax.dot_general(
        x_ref[...], w_ref[...], (((1,), (1,)), ((), ())),
        preferred_element_type=jnp.float32, precision=_PREC) + b_ref[...]


def _sempred(f_isem, W_sem, b_sem):
    return pl.pallas_call(
        _sempred_body,
        grid=(N // 512,),
        in_specs=[
            pl.BlockSpec((512, SEM_IN), lambda i: (i, 0)),
            pl.BlockSpec((SEM_OUT, SEM_IN), lambda i: (0, 0)),
            pl.BlockSpec((1, SEM_OUT), lambda i: (0, 0)),
        ],
        out_specs=pl.BlockSpec((512, SEM_OUT), lambda i: (i, 0)),
        out_shape=jax.ShapeDtypeStruct((N, SEM_OUT), jnp.float32),
        interpret=_INTERPRET,
    )(f_isem, W_sem, b_sem)


def kernel(f_sem, f_ins, W_adapt, b_adapt, W_ins, b_ins, W_sem, b_sem):
    e_ins = _embed(f_sem, f_ins, W_adapt, b_adapt.reshape(INS_IN, 1),
                   W_ins, b_ins.reshape(INS_OUT, 1))
    e_T = e_ins.transpose(0, 2, 1)
    table = f_sem.transpose(0, 2, 1)
    bs = b_sem.reshape(1, SEM_OUT)
    pms = []
    for b in range(B):
        nn = _topk(e_T[b:b + 1], e_ins[b:b + 1])
        f_isem = _gather_max(nn.reshape(N * KP), table[b])
        pms.append(_sempred(f_isem, W_sem, bs))
    pm = jnp.stack(pms)
    return (pm.transpose(0, 2, 1), e_ins)


# speedup vs baseline: 1.1764x; 1.1764x over previous
import functools

import jax
import jax.numpy as jnp
from jax import lax
from jax.experimental import pallas as pl
from jax.experimental.pallas import tpu as pltpu
from jax.experimental.pallas import tpu_sc as plsc

_INTERPRET = False

B, N, K = 4, 4096, 30
KP = 32
SEM_IN = 128
INS_IN = 128
INS_OUT = 5
SEM_OUT = 13
BN = B * N

RB = 256
CA = 512

_PREC = None


def _embed_body(fs_ref, fi_ref, wa_ref, ba_ref, wi_ref, bi_ref, e_ref):
    fs = fs_ref[0]
    fi = fi_ref[0]
    ad = jnp.maximum(
        jax.lax.dot_general(wa_ref[...], fs, (((1,), (0,)), ((), ())),
                            preferred_element_type=jnp.float32, precision=_PREC)
        + ba_ref[...], 0.0)
    fsins = fi + ad
    e = jax.lax.dot_general(wi_ref[...], fsins, (((1,), (0,)), ((), ())),
                            preferred_element_type=jnp.float32, precision=_PREC)
    e_ref[0] = e + bi_ref[...]


def _embed(f_sem, f_ins, W_adapt, b_adapt, W_ins, b_ins):
    return pl.pallas_call(
        _embed_body,
        grid=(B, N // CA),
        in_specs=[
            pl.BlockSpec((1, SEM_IN, CA), lambda b, j: (b, 0, j)),
            pl.BlockSpec((1, INS_IN, CA), lambda b, j: (b, 0, j)),
            pl.BlockSpec((INS_IN, SEM_IN), lambda b, j: (0, 0)),
            pl.BlockSpec((INS_IN, 1), lambda b, j: (0, 0)),
            pl.BlockSpec((INS_OUT, INS_IN), lambda b, j: (0, 0)),
            pl.BlockSpec((INS_OUT, 1), lambda b, j: (0, 0)),
        ],
        out_specs=pl.BlockSpec((1, INS_OUT, CA), lambda b, j: (b, 0, j)),
        out_shape=jax.ShapeDtypeStruct((B, INS_OUT, N), jnp.float32),
        interpret=_INTERPRET,
    )(f_sem, f_ins, W_adapt, b_adapt, W_ins, b_ins)


def _topk_body(eT_ref, e_ref, idx_ref):
    eq = eT_ref[0]
    ef = e_ref[0]
    sq = jnp.sum(ef * ef, axis=0, keepdims=True)
    sq_q = jnp.sum(eq * eq, axis=1, keepdims=True)
    inner = jax.lax.dot_general(eq, ef, (((1,), (0,)), ((), ())),
                                preferred_element_type=jnp.float32,
                                precision=_PREC)
    score = (sq_q - 2.0 * inner) + sq
    lanes = jax.lax.broadcasted_iota(jnp.int32, (RB, N), 1)
    j = pl.program_id(0)
    idx0 = (jax.lax.broadcasted_iota(jnp.int32, (RB, 1), 0) + j * RB)
    idx_ref[0, :, pl.ds(0, 1)] = idx0
    score = jnp.where(lanes == idx0, jnp.inf, score)
    for k in range(1, K):
        idx = jnp.argmin(score, axis=1, keepdims=True).astype(jnp.int32)
        idx_ref[0, :, pl.ds(k, 1)] = idx
        if k < K - 1:
            score = jnp.where(lanes == idx, jnp.inf, score)
    idx_ref[0, :, pl.ds(K, 1)] = idx0
    idx_ref[0, :, pl.ds(K + 1, 1)] = idx0


def _topk(e_T_b, e_ins_b):
    return pl.pallas_call(
        _topk_body,
        grid=(N // RB,),
        in_specs=[
            pl.BlockSpec((1, RB, INS_OUT), lambda j: (0, j, 0)),
            pl.BlockSpec((1, INS_OUT, N), lambda j: (0, 0, 0)),
        ],
        out_specs=pl.BlockSpec((1, RB, KP), lambda j: (0, j, 0)),
        out_shape=jax.ShapeDtypeStruct((1, N, KP), jnp.int32),
        interpret=_INTERPRET,
    )(e_T_b, e_ins_b)


NW = 32
QW = N // NW
QC = 4
NCHUNK = QW // QC


def _gather_max_body(idx_hbm, table_hbm, out_hbm,
                     idx0, idx1, rows0, rows1, out_v, sem0, sem1):
    wid = lax.axis_index("s") * 2 + lax.axis_index("c")
    qbase = wid * QW
    idx_v = (idx0, idx1)
    rows_v = (rows0, rows1)
    sems = (sem0, sem1)

    def issue(t, slot):
        off = (qbase + t * QC) * KP
        pltpu.sync_copy(idx_hbm.at[pl.ds(off, QC * KP)], idx_v[slot])
        pltpu.async_copy(table_hbm.at[idx_v[slot]], rows_v[slot], sems[slot])

    def compute(t, slot):
        pltpu.make_async_copy(table_hbm.at[idx_v[slot]], rows_v[slot],
                              sems[slot]).wait()
        rv = rows_v[slot]
        for q in range(QC):
            r0 = q * KP

            def body(r, accs):
                rr = r0 + r * 4
                for d in range(4):
                    accs = tuple(
                        jnp.maximum(accs[j], rv[rr + d, pl.ds(j * 16, 16)])
                        for j in range(8))
                return accs

            accs = tuple(rv[r0, pl.ds(j * 16, 16)] for j in range(8))
            accs = tuple(
                jnp.maximum(accs[j], rv[r0 + 1, pl.ds(j * 16, 16)])
                for j in range(8))
            accs = tuple(
                jnp.maximum(accs[j], rv[r0 + 2, pl.ds(j * 16, 16)])
                for j in range(8))
            accs = tuple(
                jnp.maximum(accs[j], rv[r0 + 3, pl.ds(j * 16, 16)])
                for j in range(8))
            accs = lax.fori_loop(1, KP // 4, body, accs)
            qq = t * QC + q
            for j in range(8):
                out_v[qq, pl.ds(j * 16, 16)] = accs[j]

    issue(0, 0)

    def pair(tt, _):
        t0 = tt * 2
        issue(t0 + 1, 1)
        compute(t0, 0)

        @pl.when(tt < NCHUNK // 2 - 1)
        def _():
            issue(t0 + 2, 0)

        compute(t0 + 1, 1)
        return 0

    lax.fori_loop(0, NCHUNK // 2, pair, 0)
    pltpu.sync_copy(out_v, out_hbm.at[pl.ds(qbase, QW)])


@functools.cache
def _gather_max_fn():
    return pl.kernel(
        _gather_max_body,
        out_type=jax.ShapeDtypeStruct((N, SEM_IN), jnp.float32),
        mesh=plsc.VectorSubcoreMesh(core_axis_name="c", subcore_axis_name="s",
                                    num_cores=2, num_subcores=16),
        scratch_types=[
            pltpu.VMEM((QC * KP,), jnp.int32),
            pltpu.VMEM((QC * KP,), jnp.int32),
            pltpu.VMEM((QC * KP, SEM_IN), jnp.float32),
            pltpu.VMEM((QC * KP, SEM_IN), jnp.float32),
            pltpu.VMEM((QW, SEM_IN), jnp.float32),
            pltpu.SemaphoreType.DMA,
            pltpu.SemaphoreType.DMA,
        ],
    )


def _gather_max(idx_flat, table):
    return _gather_max_fn()(idx_flat, table)


def _sempred_body(x_ref, w_ref, b_ref, o_ref):
    o_ref[...] = jax.lax.dot_general(
        x_ref[...], w_ref[...], (((1,), (1,)), ((), ())),
        preferred_element_type=jnp.float32, precision=_PREC) + b_ref[...]


def _sempred(f_isem, W_sem, b_sem):
    return pl.pallas_call(
        _sempred_body,
        grid=(N // 512,),
        in_specs=[
            pl.BlockSpec((512, SEM_IN), lambda i: (i, 0)),
            pl.BlockSpec((SEM_OUT, SEM_IN), lambda i: (0, 0)),
            pl.BlockSpec((1, SEM_OUT), lambda i: (0, 0)),
        ],
        out_specs=pl.BlockSpec((512, SEM_OUT), lambda i: (i, 0)),
        out_shape=jax.ShapeDtypeStruct((N, SEM_OUT), jnp.float32),
        interpret=_INTERPRET,
    )(f_isem, W_sem, b_sem)


def kernel(f_sem, f_ins, W_adapt, b_adapt, W_ins, b_ins, W_sem, b_sem):
    e_ins = _embed(f_sem, f_ins, W_adapt, b_adapt.reshape(INS_IN, 1),
                   W_ins, b_ins.reshape(INS_OUT, 1))
    e_T = e_ins.transpose(0, 2, 1)
    table = f_sem.transpose(0, 2, 1)
    bs = b_sem.reshape(1, SEM_OUT)
    pms = []
    for b in range(B):
        nn = _topk(e_T[b:b + 1], e_ins[b:b + 1])
        f_isem = _gather_max(nn.reshape(N * KP), table[b])
        pms.append(_sempred(f_isem, W_sem, bs))
    pm = jnp.stack(pms)
    return (pm.transpose(0, 2, 1), e_ins)


# speedup vs baseline: 1.1768x; 1.0004x over previous
import functools

import jax
import jax.numpy as jnp
from jax import lax
from jax.experimental import pallas as pl
from jax.experimental.pallas import tpu as pltpu
from jax.experimental.pallas import tpu_sc as plsc

B, N, K = 4, 4096, 30
KP = 32
SEM_IN = 128
INS_IN = 128
INS_OUT = 5
SEM_OUT = 13
BN = B * N

RB = 256
CA = 512

_PREC = None


def _embed_body(fs_ref, fi_ref, wa_ref, ba_ref, wi_ref, bi_ref, e_ref):
    fs = fs_ref[0]
    fi = fi_ref[0]
    ad = jnp.maximum(
        jax.lax.dot_general(wa_ref[...], fs, (((1,), (0,)), ((), ())),
                            preferred_element_type=jnp.float32, precision=_PREC)
        + ba_ref[...], 0.0)
    fsins = fi + ad
    e = jax.lax.dot_general(wi_ref[...], fsins, (((1,), (0,)), ((), ())),
                            preferred_element_type=jnp.float32, precision=_PREC)
    e_ref[0] = e + bi_ref[...]


def _embed(f_sem, f_ins, W_adapt, b_adapt, W_ins, b_ins):
    return pl.pallas_call(
        _embed_body,
        grid=(B, N // CA),
        in_specs=[
            pl.BlockSpec((1, SEM_IN, CA), lambda b, j: (b, 0, j)),
            pl.BlockSpec((1, INS_IN, CA), lambda b, j: (b, 0, j)),
            pl.BlockSpec((INS_IN, SEM_IN), lambda b, j: (0, 0)),
            pl.BlockSpec((INS_IN, 1), lambda b, j: (0, 0)),
            pl.BlockSpec((INS_OUT, INS_IN), lambda b, j: (0, 0)),
            pl.BlockSpec((INS_OUT, 1), lambda b, j: (0, 0)),
        ],
        out_specs=pl.BlockSpec((1, INS_OUT, CA), lambda b, j: (b, 0, j)),
        out_shape=jax.ShapeDtypeStruct((B, INS_OUT, N), jnp.float32),
    )(f_sem, f_ins, W_adapt, b_adapt, W_ins, b_ins)


def _topk_body(eT_ref, e_ref, idx_ref):
    eq = eT_ref[0]
    ef = e_ref[0]
    sq = jnp.sum(ef * ef, axis=0, keepdims=True)
    sq_q = jnp.sum(eq * eq, axis=1, keepdims=True)
    inner = jax.lax.dot_general(eq, ef, (((1,), (0,)), ((), ())),
                                preferred_element_type=jnp.float32,
                                precision=_PREC)
    score = (sq_q - 2.0 * inner) + sq
    lanes = jax.lax.broadcasted_iota(jnp.int32, (RB, N), 1)
    j = pl.program_id(0)
    idx0 = (jax.lax.broadcasted_iota(jnp.int32, (RB, 1), 0) + j * RB)
    idx_ref[0, :, pl.ds(0, 1)] = idx0
    score = jnp.where(lanes == idx0, jnp.inf, score)
    for k in range(1, K):
        idx = jnp.argmin(score, axis=1, keepdims=True).astype(jnp.int32)
        idx_ref[0, :, pl.ds(k, 1)] = idx
        if k < K - 1:
            score = jnp.where(lanes == idx, jnp.inf, score)
    idx_ref[0, :, pl.ds(K, 1)] = idx0
    idx_ref[0, :, pl.ds(K + 1, 1)] = idx0


def _topk(e_T_b, e_ins_b):
    return pl.pallas_call(
        _topk_body,
        grid=(N // RB,),
        in_specs=[
            pl.BlockSpec((1, RB, INS_OUT), lambda j: (0, j, 0)),
            pl.BlockSpec((1, INS_OUT, N), lambda j: (0, 0, 0)),
        ],
        out_specs=pl.BlockSpec((1, RB, KP), lambda j: (0, j, 0)),
        out_shape=jax.ShapeDtypeStruct((1, N, KP), jnp.int32),
    )(e_T_b, e_ins_b)


NW = 32
QW = N // NW
QC = 4
NCHUNK = QW // QC


def _gather_max_body(idx_hbm, table_hbm, out_hbm,
                     idx0, idx1, rows0, rows1, out_v, sem0, sem1):
    wid = lax.axis_index("s") * 2 + lax.axis_index("c")
    qbase = wid * QW
    idx_v = (idx0, idx1)
    rows_v = (rows0, rows1)
    sems = (sem0, sem1)

    def issue(t, slot):
        off = (qbase + t * QC) * KP
        pltpu.sync_copy(idx_hbm.at[pl.ds(off, QC * KP)], idx_v[slot])
        pltpu.async_copy(table_hbm.at[idx_v[slot]], rows_v[slot], sems[slot])

    def compute(t, slot):
        pltpu.make_async_copy(table_hbm.at[idx_v[slot]], rows_v[slot],
                              sems[slot]).wait()
        rv = rows_v[slot]
        for q in range(QC):
            r0 = q * KP

            def body(r, accs):
                rr = r0 + r * 4
                for d in range(4):
                    accs = tuple(
                        jnp.maximum(accs[j], rv[rr + d, pl.ds(j * 16, 16)])
                        for j in range(8))
                return accs

            accs = tuple(rv[r0, pl.ds(j * 16, 16)] for j in range(8))
            accs = tuple(
                jnp.maximum(accs[j], rv[r0 + 1, pl.ds(j * 16, 16)])
                for j in range(8))
            accs = tuple(
                jnp.maximum(accs[j], rv[r0 + 2, pl.ds(j * 16, 16)])
                for j in range(8))
            accs = tuple(
                jnp.maximum(accs[j], rv[r0 + 3, pl.ds(j * 16, 16)])
                for j in range(8))
            accs = lax.fori_loop(1, KP // 4, body, accs)
            qq = t * QC + q
            for j in range(8):
                out_v[qq, pl.ds(j * 16, 16)] = accs[j]

    issue(0, 0)

    def pair(tt, _):
        t0 = tt * 2
        issue(t0 + 1, 1)
        compute(t0, 0)

        @pl.when(tt < NCHUNK // 2 - 1)
        def _():
            issue(t0 + 2, 0)

        compute(t0 + 1, 1)
        return 0

    lax.fori_loop(0, NCHUNK // 2, pair, 0)
    pltpu.sync_copy(out_v, out_hbm.at[pl.ds(qbase, QW)])


@functools.cache
def _gather_max_fn():
    return pl.kernel(
        _gather_max_body,
        out_type=jax.ShapeDtypeStruct((N, SEM_IN), jnp.float32),
        mesh=plsc.VectorSubcoreMesh(core_axis_name="c", subcore_axis_name="s",
                                    num_cores=2, num_subcores=16),
        scratch_types=[
            pltpu.VMEM((QC * KP,), jnp.int32),
            pltpu.VMEM((QC * KP,), jnp.int32),
            pltpu.VMEM((QC * KP, SEM_IN), jnp.float32),
            pltpu.VMEM((QC * KP, SEM_IN), jnp.float32),
            pltpu.VMEM((QW, SEM_IN), jnp.float32),
            pltpu.SemaphoreType.DMA,
            pltpu.SemaphoreType.DMA,
        ],
    )


def _gather_max(idx_flat, table):
    return _gather_max_fn()(idx_flat, table)


def _sempred_body(x_ref, w_ref, b_ref, o_ref):
    o_ref[...] = jax.lax.dot_general(
        x_ref[...], w_ref[...], (((1,), (1,)), ((), ())),
        preferred_element_type=jnp.float32, precision=_PREC) + b_ref[...]


def _sempred(f_isem, W_sem, b_sem):
    return pl.pallas_call(
        _sempred_body,
        grid=(N // 512,),
        in_specs=[
            pl.BlockSpec((512, SEM_IN), lambda i: (i, 0)),
            pl.BlockSpec((SEM_OUT, SEM_IN), lambda i: (0, 0)),
            pl.BlockSpec((1, SEM_OUT), lambda i: (0, 0)),
        ],
        out_specs=pl.BlockSpec((512, SEM_OUT), lambda i: (i, 0)),
        out_shape=jax.ShapeDtypeStruct((N, SEM_OUT), jnp.float32),
    )(f_isem, W_sem, b_sem)


def kernel(f_sem, f_ins, W_adapt, b_adapt, W_ins, b_ins, W_sem, b_sem):
    e_ins = _embed(f_sem, f_ins, W_adapt, b_adapt.reshape(INS_IN, 1),
                   W_ins, b_ins.reshape(INS_OUT, 1))
    e_T = e_ins.transpose(0, 2, 1)
    table = f_sem.transpose(0, 2, 1)
    bs = b_sem.reshape(1, SEM_OUT)
    pms = []
    for b in range(B):
        nn = _topk(e_T[b:b + 1], e_ins[b:b + 1])
        f_isem = _gather_max(nn.reshape(N * KP), table[b])
        pms.append(_sempred(f_isem, W_sem, bs))
    pm = jnp.stack(pms)
    return (pm.transpose(0, 2, 1), e_ins)


# speedup vs baseline: 1.1771x; 1.0003x over previous
import functools

import jax
import jax.numpy as jnp
from jax import lax
from jax.experimental import pallas as pl
from jax.experimental.pallas import tpu as pltpu
from jax.experimental.pallas import tpu_sc as plsc

B, N, K = 4, 4096, 30
KP = 32
SEM_IN = 128
INS_IN = 128
INS_OUT = 5
SEM_OUT = 13
BN = B * N

RB = 256
CA = 512

_PREC = None


def _embed_body(fs_ref, fi_ref, wa_ref, ba_ref, wi_ref, bi_ref, e_ref,
                ft_ref):
    fs = fs_ref[0]
    fi = fi_ref[0]
    ft_ref[0] = jnp.swapaxes(fs, 0, 1)
    ad = jnp.maximum(
        jax.lax.dot_general(wa_ref[...], fs, (((1,), (0,)), ((), ())),
                            preferred_element_type=jnp.float32, precision=_PREC)
        + ba_ref[...], 0.0)
    fsins = fi + ad
    e = jax.lax.dot_general(wi_ref[...], fsins, (((1,), (0,)), ((), ())),
                            preferred_element_type=jnp.float32, precision=_PREC)
    e_ref[0] = e + bi_ref[...]


def _embed(f_sem, f_ins, W_adapt, b_adapt, W_ins, b_ins):
    return pl.pallas_call(
        _embed_body,
        grid=(B, N // CA),
        in_specs=[
            pl.BlockSpec((1, SEM_IN, CA), lambda b, j: (b, 0, j)),
            pl.BlockSpec((1, INS_IN, CA), lambda b, j: (b, 0, j)),
            pl.BlockSpec((INS_IN, SEM_IN), lambda b, j: (0, 0)),
            pl.BlockSpec((INS_IN, 1), lambda b, j: (0, 0)),
            pl.BlockSpec((INS_OUT, INS_IN), lambda b, j: (0, 0)),
            pl.BlockSpec((INS_OUT, 1), lambda b, j: (0, 0)),
        ],
        out_specs=[
            pl.BlockSpec((1, INS_OUT, CA), lambda b, j: (b, 0, j)),
            pl.BlockSpec((1, CA, SEM_IN), lambda b, j: (b, j, 0)),
        ],
        out_shape=[
            jax.ShapeDtypeStruct((B, INS_OUT, N), jnp.float32),
            jax.ShapeDtypeStruct((B, N, SEM_IN), jnp.float32),
        ],
    )(f_sem, f_ins, W_adapt, b_adapt, W_ins, b_ins)


def _topk_body(eT_ref, e_ref, idx_ref):
    eq = eT_ref[0]
    ef = e_ref[0]
    sq = jnp.sum(ef * ef, axis=0, keepdims=True)
    sq_q = jnp.sum(eq * eq, axis=1, keepdims=True)
    inner = jax.lax.dot_general(eq, ef, (((1,), (0,)), ((), ())),
                                preferred_element_type=jnp.float32,
                                precision=_PREC)
    score = (sq_q - 2.0 * inner) + sq
    lanes = jax.lax.broadcasted_iota(jnp.int32, (RB, N), 1)
    j = pl.program_id(0)
    idx0 = (jax.lax.broadcasted_iota(jnp.int32, (RB, 1), 0) + j * RB)
    idx_ref[0, :, pl.ds(0, 1)] = idx0
    score = jnp.where(lanes == idx0, jnp.inf, score)
    for k in range(1, K):
        idx = jnp.argmin(score, axis=1, keepdims=True).astype(jnp.int32)
        idx_ref[0, :, pl.ds(k, 1)] = idx
        if k < K - 1:
            score = jnp.where(lanes == idx, jnp.inf, score)
    idx_ref[0, :, pl.ds(K, 1)] = idx0
    idx_ref[0, :, pl.ds(K + 1, 1)] = idx0


def _topk(e_T_b, e_ins_b):
    return pl.pallas_call(
        _topk_body,
        grid=(N // RB,),
        in_specs=[
            pl.BlockSpec((1, RB, INS_OUT), lambda j: (0, j, 0)),
            pl.BlockSpec((1, INS_OUT, N), lambda j: (0, 0, 0)),
        ],
        out_specs=pl.BlockSpec((1, RB, KP), lambda j: (0, j, 0)),
        out_shape=jax.ShapeDtypeStruct((1, N, KP), jnp.int32),
    )(e_T_b, e_ins_b)


NW = 32
QW = N // NW
QC = 4
NCHUNK = QW // QC


def _gather_max_body(idx_hbm, table_hbm, out_hbm,
                     idx0, idx1, rows0, rows1, out_v, sem0, sem1):
    wid = lax.axis_index("s") * 2 + lax.axis_index("c")
    qbase = wid * QW
    idx_v = (idx0, idx1)
    rows_v = (rows0, rows1)
    sems = (sem0, sem1)

    def issue(t, slot):
        off = (qbase + t * QC) * KP
        pltpu.sync_copy(idx_hbm.at[pl.ds(off, QC * KP)], idx_v[slot])
        pltpu.async_copy(table_hbm.at[idx_v[slot]], rows_v[slot], sems[slot])

    def compute(t, slot):
        pltpu.make_async_copy(table_hbm.at[idx_v[slot]], rows_v[slot],
                              sems[slot]).wait()
        rv = rows_v[slot]
        for q in range(QC):
            r0 = q * KP

            def body(r, accs):
                rr = r0 + r * 4
                for d in range(4):
                    accs = tuple(
                        jnp.maximum(accs[j], rv[rr + d, pl.ds(j * 16, 16)])
                        for j in range(8))
                return accs

            accs = tuple(rv[r0, pl.ds(j * 16, 16)] for j in range(8))
            accs = tuple(
                jnp.maximum(accs[j], rv[r0 + 1, pl.ds(j * 16, 16)])
                for j in range(8))
            accs = tuple(
                jnp.maximum(accs[j], rv[r0 + 2, pl.ds(j * 16, 16)])
                for j in range(8))
            accs = tuple(
                jnp.maximum(accs[j], rv[r0 + 3, pl.ds(j * 16, 16)])
                for j in range(8))
            accs = lax.fori_loop(1, KP // 4, body, accs)
            qq = t * QC + q
            for j in range(8):
                out_v[qq, pl.ds(j * 16, 16)] = accs[j]

    issue(0, 0)

    def pair(tt, _):
        t0 = tt * 2
        issue(t0 + 1, 1)
        compute(t0, 0)

        @pl.when(tt < NCHUNK // 2 - 1)
        def _():
            issue(t0 + 2, 0)

        compute(t0 + 1, 1)
        return 0

    lax.fori_loop(0, NCHUNK // 2, pair, 0)
    pltpu.sync_copy(out_v, out_hbm.at[pl.ds(qbase, QW)])


@functools.cache
def _gather_max_fn():
    return pl.kernel(
        _gather_max_body,
        out_type=jax.ShapeDtypeStruct((N, SEM_IN), jnp.float32),
        mesh=plsc.VectorSubcoreMesh(core_axis_name="c", subcore_axis_name="s",
                                    num_cores=2, num_subcores=16),
        scratch_types=[
            pltpu.VMEM((QC * KP,), jnp.int32),
            pltpu.VMEM((QC * KP,), jnp.int32),
            pltpu.VMEM((QC * KP, SEM_IN), jnp.float32),
            pltpu.VMEM((QC * KP, SEM_IN), jnp.float32),
            pltpu.VMEM((QW, SEM_IN), jnp.float32),
            pltpu.SemaphoreType.DMA,
            pltpu.SemaphoreType.DMA,
        ],
    )


def _gather_max(idx_flat, table):
    return _gather_max_fn()(idx_flat, table)


def _sempred_body(x_ref, w_ref, b_ref, o_ref):
    o_ref[...] = jax.lax.dot_general(
        x_ref[...], w_ref[...], (((1,), (1,)), ((), ())),
        preferred_element_type=jnp.float32, precision=_PREC) + b_ref[...]


def _sempred(f_isem, W_sem, b_sem):
    return pl.pallas_call(
        _sempred_body,
        grid=(N // 512,),
        in_specs=[
            pl.BlockSpec((512, SEM_IN), lambda i: (i, 0)),
            pl.BlockSpec((SEM_OUT, SEM_IN), lambda i: (0, 0)),
            pl.BlockSpec((1, SEM_OUT), lambda i: (0, 0)),
        ],
        out_specs=pl.BlockSpec((512, SEM_OUT), lambda i: (i, 0)),
        out_shape=jax.ShapeDtypeStruct((N, SEM_OUT), jnp.float32),
    )(f_isem, W_sem, b_sem)


def kernel(f_sem, f_ins, W_adapt, b_adapt, W_ins, b_ins, W_sem, b_sem):
    e_ins, table = _embed(f_sem, f_ins, W_adapt, b_adapt.reshape(INS_IN, 1),
                          W_ins, b_ins.reshape(INS_OUT, 1))
    e_T = e_ins.transpose(0, 2, 1)
    bs = b_sem.reshape(1, SEM_OUT)
    pms = []
    for b in range(B):
        nn = _topk(e_T[b:b + 1], e_ins[b:b + 1])
        f_isem = _gather_max(nn.reshape(N * KP), table[b])
        pms.append(_sempred(f_isem, W_sem, bs))
    pm = jnp.stack(pms)
    return (pm.transpose(0, 2, 1), e_ins)
